# baseline (speedup 1.0000x reference)
import functools

import jax
import jax.numpy as jnp
from jax import lax
from jax.experimental import pallas as pl
from jax.experimental.pallas import tpu as pltpu
from jax.experimental.pallas import tpu_sc as plsc

N_NODES = 10000
N_EDGES = 320000
D = 128
DH = D // 2

NC = 2
NS = 16
BLK = 512
NBLK = 40
EPW = BLK * NBLK
E_PAD = NS * EPW
N_ROWS = 10240
RPT = N_ROWS // NS
CH = 128
DUMMY = N_NODES
NB = 1


def _sc_body(feat_hbm, src_hbm, dst_hbm, out_hbm, srcv, dstv,
             b0, b1, accum, g0, g1, s0, s1):
    bufs = (b0,)
    gsem = (g0,)
    ssem = (s0,)
    cid = lax.axis_index("c")
    sid = lax.axis_index("s")

    zero16 = jnp.zeros((16,), jnp.float32)

    def zrow(i, c):
        for j in range(DH // 16):
            b0[i, pl.ds(16 * j, 16)] = zero16
        return c

    lax.fori_loop(0, CH, zrow, 0)
    for t in range(RPT // CH):
        pltpu.sync_copy(b0.at[pl.ds(0, CH)],
                        accum.at[pl.ds(sid * RPT + t * CH, CH)])

    pltpu.sync_copy(src_hbm.at[pl.ds(sid * EPW, EPW)], srcv)
    pltpu.sync_copy(dst_hbm.at[pl.ds(sid * EPW, EPW)], dstv)

    off = cid * N_NODES

    def orow(i, c):
        sl = pl.ds(16 * i, 16)
        srcv[sl] = srcv[sl] + off
        return c

    lax.fori_loop(0, EPW // 16, orow, 0)
    plsc.subcore_barrier()

    def group(g, c):
        jj = g * NB * BLK
        cps = [pltpu.async_copy(feat_hbm.at[srcv.at[pl.ds(jj + i * BLK, BLK)]],
                                bufs[i], gsem[i])
               for i in range(NB)]
        scs = []
        for i in range(NB):
            cps[i].wait()
            scs.append(pltpu.async_copy(bufs[i],
                                        accum.at[dstv.at[pl.ds(jj + i * BLK, BLK)]],
                                        ssem[i], add=True))
        for s in scs:
            s.wait()
        return c

    lax.fori_loop(0, NBLK // NB, group, 0)
    plsc.subcore_barrier()

    for t in range(RPT // CH):
        r = sid * RPT + t * CH
        pltpu.sync_copy(accum.at[pl.ds(r, CH)], out_hbm.at[cid, pl.ds(r, CH)])


_sc_gcn = functools.partial(
    pl.kernel,
    mesh=plsc.VectorSubcoreMesh(core_axis_name="c", subcore_axis_name="s"),
    compiler_params=pltpu.CompilerParams(use_tc_tiling_on_sc=False),
    out_type=jax.ShapeDtypeStruct((NC, N_ROWS, DH), jnp.float32),
    scratch_types=[
        pltpu.VMEM((EPW,), jnp.int32),
        pltpu.VMEM((EPW,), jnp.int32),
        pltpu.VMEM((BLK, DH), jnp.float32),
        pltpu.VMEM((BLK, DH), jnp.float32),
        pltpu.VMEM_SHARED((N_ROWS, DH), jnp.float32),
        pltpu.SemaphoreType.DMA,
        pltpu.SemaphoreType.DMA,
        pltpu.SemaphoreType.DMA,
        pltpu.SemaphoreType.DMA,
    ],
)(_sc_body)


def _tc_body(p_ref, w_ref, b_ref, o_ref):
    x = jnp.concatenate([p_ref[0], p_ref[1]], axis=1)
    o_ref[...] = (
        lax.dot_general(x, w_ref[...], (((1,), (1,)), ((), ())),
                        preferred_element_type=jnp.float32)
        + b_ref[...]
    )


def _tc_linear(partials, W, b2):
    blk = 400
    return pl.pallas_call(
        _tc_body,
        grid=(N_NODES // blk,),
        in_specs=[
            pl.BlockSpec((NC, blk, DH), lambda i: (0, i, 0)),
            pl.BlockSpec((D, D), lambda i: (0, 0)),
            pl.BlockSpec((1, D), lambda i: (0, 0)),
        ],
        out_specs=pl.BlockSpec((blk, D), lambda i: (i, 0)),
        out_shape=jax.ShapeDtypeStruct((N_NODES, D), jnp.float32),
    )(partials, W, b2)


def kernel(feature, edge_index, W, b):
    ei = edge_index.astype(jnp.int32)
    pad = E_PAD - N_EDGES
    src2 = jnp.concatenate([ei[0], jnp.zeros((pad,), jnp.int32)])
    dst2 = jnp.concatenate([ei[1], jnp.full((pad,), DUMMY, jnp.int32)])
    fstk = jnp.concatenate([feature[:, :DH], feature[:, DH:]], axis=0)
    partials = _sc_gcn(fstk, src2, dst2)
    return _tc_linear(partials, W, b.reshape(1, D))


# speedup vs baseline: 1.2926x; 1.2926x over previous
import functools

import jax
import jax.numpy as jnp
from jax import lax
from jax.experimental import pallas as pl
from jax.experimental.pallas import tpu as pltpu
from jax.experimental.pallas import tpu_sc as plsc

N_NODES = 10000
N_EDGES = 320000
D = 128

NC = 2
NS = 16
NW = NC * NS
BLK = 512
NBLK = 20
EPW = BLK * NBLK
E_PAD = NW * EPW
N_ROWS = 10240
RPT = N_ROWS // NS
CH = 128
DUMMY = N_NODES


def _sc_body(feat_hbm, src_hbm, dst_hbm, out_hbm, srcv, dstv, buf, accum, sem):
    cid = lax.axis_index("c")
    sid = lax.axis_index("s")
    wid = cid * NS + sid

    zero32 = jnp.zeros((32,), jnp.bfloat16)

    def zrow(i, c):
        for j in range(D // 32):
            buf[i, pl.ds(32 * j, 32)] = zero32
        return c

    lax.fori_loop(0, CH, zrow, 0)
    for t in range(RPT // CH):
        pltpu.sync_copy(buf.at[pl.ds(0, CH)],
                        accum.at[pl.ds(sid * RPT + t * CH, CH)])

    pltpu.sync_copy(src_hbm.at[pl.ds(wid * EPW, EPW)], srcv)
    pltpu.sync_copy(dst_hbm.at[pl.ds(wid * EPW, EPW)], dstv)
    plsc.subcore_barrier()

    def body(g, c):
        jj = g * BLK
        pltpu.async_copy(feat_hbm.at[srcv.at[pl.ds(jj, BLK)]], buf, sem).wait()
        pltpu.sync_copy(buf, accum.at[dstv.at[pl.ds(jj, BLK)]], add=True)
        return c

    lax.fori_loop(0, NBLK, body, 0)
    plsc.subcore_barrier()

    for t in range(RPT // CH):
        r = sid * RPT + t * CH
        pltpu.sync_copy(accum.at[pl.ds(r, CH)], out_hbm.at[cid, pl.ds(r, CH)])


_sc_gcn = functools.partial(
    pl.kernel,
    mesh=plsc.VectorSubcoreMesh(core_axis_name="c", subcore_axis_name="s"),
    compiler_params=pltpu.CompilerParams(use_tc_tiling_on_sc=False),
    out_type=jax.ShapeDtypeStruct((NC, N_ROWS, D), jnp.bfloat16),
    scratch_types=[
        pltpu.VMEM((EPW,), jnp.int32),
        pltpu.VMEM((EPW,), jnp.int32),
        pltpu.VMEM((BLK, D), jnp.bfloat16),
        pltpu.VMEM_SHARED((N_ROWS, D), jnp.bfloat16),
        pltpu.SemaphoreType.DMA,
    ],
)(_sc_body)


def _tc_body(p_ref, w_ref, b_ref, o_ref):
    x = (p_ref[0].astype(jnp.float32) + p_ref[1].astype(jnp.float32))
    o_ref[...] = (
        lax.dot_general(x, w_ref[...], (((1,), (1,)), ((), ())),
                        preferred_element_type=jnp.float32)
        + b_ref[...]
    )


def _tc_linear(partials, W, b2):
    blk = 400
    return pl.pallas_call(
        _tc_body,
        grid=(N_NODES // blk,),
        in_specs=[
            pl.BlockSpec((NC, blk, D), lambda i: (0, i, 0)),
            pl.BlockSpec((D, D), lambda i: (0, 0)),
            pl.BlockSpec((1, D), lambda i: (0, 0)),
        ],
        out_specs=pl.BlockSpec((blk, D), lambda i: (i, 0)),
        out_shape=jax.ShapeDtypeStruct((N_NODES, D), jnp.float32),
    )(partials, W, b2)


def kernel(feature, edge_index, W, b):
    ei = edge_index.astype(jnp.int32)
    pad = E_PAD - N_EDGES
    src2 = jnp.concatenate([ei[0], jnp.zeros((pad,), jnp.int32)])
    dst2 = jnp.concatenate([ei[1], jnp.full((pad,), DUMMY, jnp.int32)])
    fb = feature.astype(jnp.bfloat16)
    partials = _sc_gcn(fb, src2, dst2)
    return _tc_linear(partials, W, b.reshape(1, D))


# speedup vs baseline: 1.3705x; 1.0603x over previous
import functools

import jax
import jax.numpy as jnp
from jax import lax
from jax.experimental import pallas as pl
from jax.experimental.pallas import tpu as pltpu
from jax.experimental.pallas import tpu_sc as plsc

N_NODES = 10000
N_EDGES = 320000
D = 128

NC = 2
NS = 16
NW = NC * NS
BLK = 512
NBLK = 20
EPW = BLK * NBLK
E_PAD = NW * EPW
N_ROWS = 10240
RPT = N_ROWS // NS
CH = 128
DUMMY = N_NODES


def _sc_body(feat_hbm, src_hbm, dst_hbm, out_hbm, srcv, dstv,
             b0, b1, accum, g0, g1, s0, s1):
    buf = b0
    cid = lax.axis_index("c")
    sid = lax.axis_index("s")
    wid = cid * NS + sid

    zero32 = jnp.zeros((32,), jnp.bfloat16)

    def zrow(i, c):
        for j in range(D // 32):
            buf[i, pl.ds(32 * j, 32)] = zero32
        return c

    lax.fori_loop(0, CH, zrow, 0)
    for t in range(RPT // CH):
        pltpu.sync_copy(buf.at[pl.ds(0, CH)],
                        accum.at[pl.ds(sid * RPT + t * CH, CH)])

    pltpu.sync_copy(src_hbm.at[pl.ds(wid * EPW, EPW)], srcv)
    pltpu.sync_copy(dst_hbm.at[pl.ds(wid * EPW, EPW)], dstv)
    plsc.subcore_barrier()

    bufs = (b0, b1)
    gsem = (g0, g1)
    ssem = (s0, s1)

    def body(g, c):
        jj = g * 2 * BLK
        cps = [pltpu.async_copy(feat_hbm.at[srcv.at[pl.ds(jj + i * BLK, BLK)]],
                                bufs[i], gsem[i])
               for i in range(2)]
        scs = []
        for i in range(2):
            cps[i].wait()
            scs.append(pltpu.async_copy(bufs[i],
                                        accum.at[dstv.at[pl.ds(jj + i * BLK, BLK)]],
                                        ssem[i], add=True))
        for s_ in scs:
            s_.wait()
        return c

    lax.fori_loop(0, NBLK // 2, body, 0)
    plsc.subcore_barrier()

    for t in range(RPT // CH):
        r = sid * RPT + t * CH
        pltpu.sync_copy(accum.at[pl.ds(r, CH)], out_hbm.at[cid, pl.ds(r, CH)])


_sc_gcn = functools.partial(
    pl.kernel,
    mesh=plsc.VectorSubcoreMesh(core_axis_name="c", subcore_axis_name="s"),
    compiler_params=pltpu.CompilerParams(use_tc_tiling_on_sc=False),
    out_type=jax.ShapeDtypeStruct((NC, N_ROWS, D), jnp.bfloat16),
    scratch_types=[
        pltpu.VMEM((EPW,), jnp.int32),
        pltpu.VMEM((EPW,), jnp.int32),
        pltpu.VMEM((BLK, D), jnp.bfloat16),
        pltpu.VMEM((BLK, D), jnp.bfloat16),
        pltpu.VMEM_SHARED((N_ROWS, D), jnp.bfloat16),
        pltpu.SemaphoreType.DMA,
        pltpu.SemaphoreType.DMA,
        pltpu.SemaphoreType.DMA,
        pltpu.SemaphoreType.DMA,
    ],
)(_sc_body)


def _tc_body(p_ref, w_ref, b_ref, o_ref):
    x = (p_ref[0].astype(jnp.float32) + p_ref[1].astype(jnp.float32))
    o_ref[...] = (
        lax.dot_general(x, w_ref[...], (((1,), (1,)), ((), ())),
                        preferred_element_type=jnp.float32)
        + b_ref[...]
    )


def _tc_linear(partials, W, b2):
    blk = 400
    return pl.pallas_call(
        _tc_body,
        grid=(N_NODES // blk,),
        in_specs=[
            pl.BlockSpec((NC, blk, D), lambda i: (0, i, 0)),
            pl.BlockSpec((D, D), lambda i: (0, 0)),
            pl.BlockSpec((1, D), lambda i: (0, 0)),
        ],
        out_specs=pl.BlockSpec((blk, D), lambda i: (i, 0)),
        out_shape=jax.ShapeDtypeStruct((N_NODES, D), jnp.float32),
    )(partials, W, b2)


def kernel(feature, edge_index, W, b):
    ei = edge_index.astype(jnp.int32)
    pad = E_PAD - N_EDGES
    src2 = jnp.concatenate([ei[0], jnp.zeros((pad,), jnp.int32)])
    dst2 = jnp.concatenate([ei[1], jnp.full((pad,), DUMMY, jnp.int32)])
    fb = feature.astype(jnp.bfloat16)
    partials = _sc_gcn(fb, src2, dst2)
    return _tc_linear(partials, W, b.reshape(1, D))


# speedup vs baseline: 2.8898x; 2.1085x over previous
import functools

import jax
import jax.numpy as jnp
from jax import lax
from jax.experimental import pallas as pl
from jax.experimental.pallas import tpu as pltpu
from jax.experimental.pallas import tpu_sc as plsc

N_NODES = 10000
N_EDGES = 320000
D = 128

NC = 2
NS = 16
NW = NC * NS
BLK = 512
NBLK = 20
EPW = BLK * NBLK
E_PAD = NW * EPW
N_ROWS = 10240
RPT = N_ROWS // NS
CH = 128
DUMMY = N_NODES


def _sc_body(feat_hbm, src_hbm, dst_hbm, out_hbm, srcv, dstv,
             b0, b1, accum, g0, g1, s0, s1):
    buf = b0
    cid = lax.axis_index("c")
    sid = lax.axis_index("s")
    wid = cid * NS + sid

    zero32 = jnp.zeros((32,), jnp.bfloat16)

    def zrow(i, c):
        for j in range(D // 32):
            buf[i, pl.ds(32 * j, 32)] = zero32
        return c

    lax.fori_loop(0, CH, zrow, 0)
    for t in range(RPT // CH):
        pltpu.sync_copy(buf.at[pl.ds(0, CH)],
                        accum.at[pl.ds(sid * RPT + t * CH, CH)])

    pltpu.sync_copy(src_hbm.at[pl.ds(wid * EPW, EPW)], srcv)
    pltpu.sync_copy(dst_hbm.at[pl.ds(wid * EPW, EPW)], dstv)
    plsc.subcore_barrier()

    bufs = (b0, b1)
    gsem = (g0, g1)
    ssem = (s0, s1)

    def body(g, c):
        jj = g * 2 * BLK
        cps = [pltpu.async_copy(feat_hbm.at[srcv.at[pl.ds(jj + i * BLK, BLK)]],
                                bufs[i], gsem[i])
               for i in range(2)]
        scs = []
        for i in range(2):
            cps[i].wait()
            scs.append(pltpu.async_copy(bufs[i],
                                        accum.at[dstv.at[pl.ds(jj + i * BLK, BLK)]],
                                        ssem[i], add=True))
        for s_ in scs:
            s_.wait()
        return c

    lax.fori_loop(0, NBLK // 2, body, 0)
    plsc.subcore_barrier()

    for t in range(RPT // CH):
        r = sid * RPT + t * CH
        pltpu.sync_copy(accum.at[pl.ds(r, CH)], out_hbm.at[cid, pl.ds(r, CH)])


_sc_gcn = functools.partial(
    pl.kernel,
    mesh=plsc.VectorSubcoreMesh(core_axis_name="c", subcore_axis_name="s"),
    compiler_params=pltpu.CompilerParams(use_tc_tiling_on_sc=False),
    out_type=jax.ShapeDtypeStruct((NC, N_ROWS, D), jnp.bfloat16),
    scratch_types=[
        pltpu.VMEM((EPW,), jnp.int32),
        pltpu.VMEM((EPW,), jnp.int32),
        pltpu.VMEM((BLK, D), jnp.bfloat16),
        pltpu.VMEM((BLK, D), jnp.bfloat16),
        pltpu.VMEM_SHARED((N_ROWS, D), jnp.bfloat16),
        pltpu.SemaphoreType.DMA,
        pltpu.SemaphoreType.DMA,
        pltpu.SemaphoreType.DMA,
        pltpu.SemaphoreType.DMA,
    ],
)(_sc_body)


def _tc_body(p_ref, w_ref, b_ref, o_ref):
    x = (p_ref[0].astype(jnp.float32) + p_ref[1].astype(jnp.float32))
    o_ref[...] = (
        lax.dot_general(x, w_ref[...], (((1,), (1,)), ((), ())),
                        preferred_element_type=jnp.float32)
        + b_ref[...]
    )


def _tc_linear(partials, W, b2):
    blk = 400
    return pl.pallas_call(
        _tc_body,
        grid=(N_NODES // blk,),
        in_specs=[
            pl.BlockSpec((NC, blk, D), lambda i: (0, i, 0)),
            pl.BlockSpec((D, D), lambda i: (0, 0)),
            pl.BlockSpec((1, D), lambda i: (0, 0)),
        ],
        out_specs=pl.BlockSpec((blk, D), lambda i: (i, 0)),
        out_shape=jax.ShapeDtypeStruct((N_NODES, D), jnp.float32),
    )(partials, W, b2)


def kernel(feature, edge_index, W, b):
    ei = edge_index.astype(jnp.int32)
    pad = E_PAD - N_EDGES
    ar = jnp.arange(pad, dtype=jnp.int32)
    src2 = jnp.concatenate([ei[0], ar % N_NODES])
    dst2 = jnp.concatenate([ei[1], DUMMY + ar % (N_ROWS - N_NODES)])
    fb = feature.astype(jnp.bfloat16)
    partials = _sc_gcn(fb, src2, dst2)
    return _tc_linear(partials, W, b.reshape(1, D))


# speedup vs baseline: 2.9320x; 1.0146x over previous
import functools

import jax
import jax.numpy as jnp
from jax import lax
from jax.experimental import pallas as pl
from jax.experimental.pallas import tpu as pltpu
from jax.experimental.pallas import tpu_sc as plsc

N_NODES = 10000
N_EDGES = 320000
D = 128

NC = 2
NS = 16
NW = NC * NS
BLK = 400
NBLK = 25
EPW = BLK * NBLK
E_PAD = NW * EPW
N_ROWS = 10240
RPT = N_ROWS // NS
CH = 128
DUMMY = N_NODES


def _sc_body(feat_hbm, src_hbm, dst_hbm, out_hbm, srcv, dstv,
             b0, b1, accum, g0, g1, s0, s1):
    buf = b0
    cid = lax.axis_index("c")
    sid = lax.axis_index("s")
    wid = cid * NS + sid

    zero32 = jnp.zeros((32,), jnp.bfloat16)

    def zrow(i, c):
        for j in range(D // 32):
            buf[i, pl.ds(32 * j, 32)] = zero32
        return c

    lax.fori_loop(0, CH, zrow, 0)
    for t in range(RPT // CH):
        pltpu.sync_copy(buf.at[pl.ds(0, CH)],
                        accum.at[pl.ds(sid * RPT + t * CH, CH)])

    pltpu.sync_copy(src_hbm.at[pl.ds(wid * EPW, EPW)], srcv)
    pltpu.sync_copy(dst_hbm.at[pl.ds(wid * EPW, EPW)], dstv)
    plsc.subcore_barrier()

    bufs = (b0, b1)
    gsem = (g0, g1)
    ssem = (s0, s1)

    def body(g, c):
        jj = g * 2 * BLK
        cps = [pltpu.async_copy(feat_hbm.at[srcv.at[pl.ds(jj + i * BLK, BLK)]],
                                bufs[i], gsem[i])
               for i in range(2)]
        scs = []
        for i in range(2):
            cps[i].wait()
            scs.append(pltpu.async_copy(bufs[i],
                                        accum.at[dstv.at[pl.ds(jj + i * BLK, BLK)]],
                                        ssem[i], add=True))
        for s_ in scs:
            s_.wait()
        return c

    lax.fori_loop(0, NBLK // 2, body, 0)

    jl = (NBLK - 1) * BLK
    pltpu.async_copy(feat_hbm.at[srcv.at[pl.ds(jl, BLK)]], b0, g0).wait()
    pltpu.async_copy(b0, accum.at[dstv.at[pl.ds(jl, BLK)]], s0, add=True).wait()
    plsc.subcore_barrier()

    for t in range(RPT // CH):
        r = sid * RPT + t * CH
        pltpu.sync_copy(accum.at[pl.ds(r, CH)], out_hbm.at[cid, pl.ds(r, CH)])


_sc_gcn = functools.partial(
    pl.kernel,
    mesh=plsc.VectorSubcoreMesh(core_axis_name="c", subcore_axis_name="s"),
    compiler_params=pltpu.CompilerParams(use_tc_tiling_on_sc=False),
    out_type=jax.ShapeDtypeStruct((NC, N_ROWS, D), jnp.bfloat16),
    scratch_types=[
        pltpu.VMEM((EPW,), jnp.int32),
        pltpu.VMEM((EPW,), jnp.int32),
        pltpu.VMEM((BLK, D), jnp.bfloat16),
        pltpu.VMEM((BLK, D), jnp.bfloat16),
        pltpu.VMEM_SHARED((N_ROWS, D), jnp.bfloat16),
        pltpu.SemaphoreType.DMA,
        pltpu.SemaphoreType.DMA,
        pltpu.SemaphoreType.DMA,
        pltpu.SemaphoreType.DMA,
    ],
)(_sc_body)


def _tc_body(p_ref, w_ref, b_ref, o_ref):
    x = (p_ref[0].astype(jnp.float32) + p_ref[1].astype(jnp.float32))
    o_ref[...] = (
        lax.dot_general(x, w_ref[...], (((1,), (1,)), ((), ())),
                        preferred_element_type=jnp.float32)
        + b_ref[...]
    )


def _tc_linear(partials, W, b2):
    blk = 400
    return pl.pallas_call(
        _tc_body,
        grid=(N_NODES // blk,),
        in_specs=[
            pl.BlockSpec((NC, blk, D), lambda i: (0, i, 0)),
            pl.BlockSpec((D, D), lambda i: (0, 0)),
            pl.BlockSpec((1, D), lambda i: (0, 0)),
        ],
        out_specs=pl.BlockSpec((blk, D), lambda i: (i, 0)),
        out_shape=jax.ShapeDtypeStruct((N_NODES, D), jnp.float32),
    )(partials, W, b2)


def kernel(feature, edge_index, W, b):
    ei = edge_index.astype(jnp.int32)
    fb = feature.astype(jnp.bfloat16)
    partials = _sc_gcn(fb, ei[0], ei[1])
    return _tc_linear(partials, W, b.reshape(1, D))


# speedup vs baseline: 3.1364x; 1.0697x over previous
import functools

import jax
import jax.numpy as jnp
from jax import lax
from jax.experimental import pallas as pl
from jax.experimental.pallas import tpu as pltpu
from jax.experimental.pallas import tpu_sc as plsc

N_NODES = 10000
N_EDGES = 320000
D = 128

NC = 2
NS = 16
NW = NC * NS
BLK = 400
NBLK = 25
EPW = BLK * NBLK
E_PAD = NW * EPW
N_ROWS = 10240
RPT = N_ROWS // NS
CH = 128
DUMMY = N_NODES


def _sc_body(feat_hbm, src_hbm, dst_hbm, out_hbm, srcv, dstv,
             b0, b1, accum, g0, g1, s0, s1):
    buf = b0
    cid = lax.axis_index("c")
    sid = lax.axis_index("s")
    wid = cid * NS + sid

    zero32 = jnp.zeros((32,), jnp.bfloat16)

    def zrow(i, c):
        for j in range(D // 32):
            buf[i, pl.ds(32 * j, 32)] = zero32
        return c

    lax.fori_loop(0, CH, zrow, 0)
    for t in range(RPT // CH):
        pltpu.sync_copy(buf.at[pl.ds(0, CH)],
                        accum.at[pl.ds(sid * RPT + t * CH, CH)])

    pltpu.sync_copy(src_hbm.at[pl.ds(wid * EPW, EPW)], srcv)
    pltpu.sync_copy(dst_hbm.at[pl.ds(wid * EPW, EPW)], dstv)
    plsc.subcore_barrier()

    bufs = (b0, b1)
    gsem = (g0, g1)
    ssem = (s0, s1)

    def body(g, c):
        jj = g * 2 * BLK
        cps = [pltpu.async_copy(feat_hbm.at[srcv.at[pl.ds(jj + i * BLK, BLK)]],
                                bufs[i], gsem[i])
               for i in range(2)]
        scs = []
        for i in range(2):
            cps[i].wait()
            scs.append(pltpu.async_copy(bufs[i],
                                        accum.at[dstv.at[pl.ds(jj + i * BLK, BLK)]],
                                        ssem[i], add=True))
        for s_ in scs:
            s_.wait()
        return c

    lax.fori_loop(0, NBLK // 2, body, 0)

    jl = (NBLK - 1) * BLK
    pltpu.async_copy(feat_hbm.at[srcv.at[pl.ds(jl, BLK)]], b0, g0).wait()
    pltpu.async_copy(b0, accum.at[dstv.at[pl.ds(jl, BLK)]], s0, add=True).wait()
    plsc.subcore_barrier()

    for t in range(RPT // CH):
        r = sid * RPT + t * CH
        pltpu.sync_copy(accum.at[pl.ds(r, CH)], out_hbm.at[cid, pl.ds(r, CH)])


_sc_gcn = functools.partial(
    pl.kernel,
    mesh=plsc.VectorSubcoreMesh(core_axis_name="c", subcore_axis_name="s"),
    compiler_params=pltpu.CompilerParams(use_tc_tiling_on_sc=False),
    out_type=jax.ShapeDtypeStruct((NC, N_ROWS, D), jnp.bfloat16),
    scratch_types=[
        pltpu.VMEM((EPW,), jnp.int32),
        pltpu.VMEM((EPW,), jnp.int32),
        pltpu.VMEM((BLK, D), jnp.bfloat16),
        pltpu.VMEM((BLK, D), jnp.bfloat16),
        pltpu.VMEM_SHARED((N_ROWS, D), jnp.bfloat16),
        pltpu.SemaphoreType.DMA,
        pltpu.SemaphoreType.DMA,
        pltpu.SemaphoreType.DMA,
        pltpu.SemaphoreType.DMA,
    ],
)(_sc_body)


def _tc_body(p_ref, w_ref, b_ref, o_ref):
    x = (p_ref[0].astype(jnp.float32) + p_ref[1].astype(jnp.float32))
    o_ref[...] = (
        lax.dot_general(x, w_ref[...], (((1,), (1,)), ((), ())),
                        preferred_element_type=jnp.float32)
        + b_ref[...]
    )


def _tc_linear(partials, W, b2):
    blk = 2000
    return pl.pallas_call(
        _tc_body,
        grid=(N_NODES // blk,),
        in_specs=[
            pl.BlockSpec((NC, blk, D), lambda i: (0, i, 0)),
            pl.BlockSpec((D, D), lambda i: (0, 0)),
            pl.BlockSpec((1, D), lambda i: (0, 0)),
        ],
        out_specs=pl.BlockSpec((blk, D), lambda i: (i, 0)),
        out_shape=jax.ShapeDtypeStruct((N_NODES, D), jnp.float32),
    )(partials, W, b2)


def kernel(feature, edge_index, W, b):
    ei = edge_index.astype(jnp.int32)
    fb = feature.astype(jnp.bfloat16)
    partials = _sc_gcn(fb, ei[0], ei[1])
    return _tc_linear(partials, W, b.reshape(1, D))
